# baseline (speedup 1.0000x reference)
import functools

import jax
import jax.numpy as jnp
from jax import lax
from jax.experimental import pallas as pl
from jax.experimental.pallas import tpu as pltpu
from jax.experimental.pallas import tpu_sc as plsc

N = 10000
E = 320000
R = 16
NB = 16
D = 128
BN = 1000

NTILES = 32
CH = 80
EPT = 10240
EPAD = NTILES * EPT
EPC = EPAD // 16
NRP = 163840
NACC = 10240
NPAD = NTILES * 320

_MESH = plsc.VectorSubcoreMesh(core_axis_name="c", subcore_axis_name="s")


def _zero_vmem_1d(buf, nvec):
    z = jnp.zeros((16,), jnp.float32)

    def body(i, _):
        buf[pl.ds(i * 16, 16)] = z
        return 0

    lax.fori_loop(0, nvec, body, 0)


def _zero_vmem_2d(buf, nrows):
    z = jnp.zeros((16,), jnp.float32)

    def body(i, _):
        for dd in range(D // 16):
            buf[i, pl.ds(dd * 16, 16)] = z
        return 0

    lax.fori_loop(0, nrows, body, 0)


@functools.partial(
    pl.kernel,
    out_type=jax.ShapeDtypeStruct((NPAD, D), jnp.float32),
    mesh=_MESH,
    scratch_types=[
        pltpu.VMEM((80,), jnp.int32),
        pltpu.VMEM((80, D), jnp.float32),
        pltpu.SemaphoreType.DMA,
    ],
)
def _sc_x0(emb_hbm, ids_hbm, out_hbm, idxb, rows, sem):
    c = lax.axis_index("c")
    s = lax.axis_index("s")
    wid = c * 16 + s

    def chunk(j, _):
        base = wid * 320 + j * 80
        pltpu.sync_copy(ids_hbm.at[pl.ds(base, 80)], idxb)
        pltpu.async_copy(emb_hbm.at[idxb], rows, sem).wait()
        pltpu.sync_copy(rows, out_hbm.at[pl.ds(base, 80)])
        return 0

    lax.fori_loop(0, 4, chunk, 0)


@functools.partial(
    pl.kernel,
    out_type=jax.ShapeDtypeStruct((EPAD,), jnp.float32),
    mesh=_MESH,
    scratch_types=[
        pltpu.VMEM_SHARED((NRP,), jnp.float32),
        pltpu.VMEM((CH,), jnp.int32),
        pltpu.VMEM((CH,), jnp.float32),
        pltpu.VMEM((2048,), jnp.float32),
        pltpu.SemaphoreType.DMA,
    ],
)
def _sc_weights(seg_hbm, w_hbm, cnt, segb, wb, stage, sem):
    c = lax.axis_index("c")
    s = lax.axis_index("s")
    wid = c * 16 + s

    def ones_body(i, _):
        wb[pl.ds(i * 16, 16)] = jnp.full((16,), 1.0, jnp.float32)
        return 0

    lax.fori_loop(0, CH // 16, ones_body, 0)

    _zero_vmem_1d(stage, 128)

    def zchunk(j, _):
        pltpu.sync_copy(stage, cnt.at[pl.ds(s * (NRP // 16) + j * 2048, 2048)])
        return 0

    lax.fori_loop(0, NRP // 16 // 2048, zchunk, 0)
    plsc.subcore_barrier()

    def cchunk(k, _):
        pltpu.sync_copy(seg_hbm.at[pl.ds(s * EPC + k * CH, CH)], segb)
        pltpu.sync_copy(wb, cnt.at[segb], add=True)
        return 0

    lax.fori_loop(0, EPC // CH, cchunk, 0)
    plsc.subcore_barrier()

    def ichunk(j, _):
        base = s * (NRP // 16) + j * 2048
        pltpu.sync_copy(cnt.at[pl.ds(base, 2048)], stage)

        def inv(i, _):
            v = stage[pl.ds(i * 16, 16)]
            stage[pl.ds(i * 16, 16)] = 1.0 / jnp.maximum(v, 1.0)
            return 0

        lax.fori_loop(0, 128, inv, 0)
        pltpu.sync_copy(stage, cnt.at[pl.ds(base, 2048)])
        return 0

    lax.fori_loop(0, NRP // 16 // 2048, ichunk, 0)
    plsc.subcore_barrier()

    def gchunk(k, _):
        base = wid * EPT + k * CH
        pltpu.sync_copy(seg_hbm.at[pl.ds(base, CH)], segb)
        pltpu.async_copy(cnt.at[segb], wb, sem).wait()
        pltpu.sync_copy(wb, w_hbm.at[pl.ds(base, CH)])
        return 0

    lax.fori_loop(0, EPT // CH, gchunk, 0)


NCHT = EPT // CH
NCH_A = 105
NCH_B = 2 * NCHT - 105
RBYTES = CH * D * 4


@functools.partial(
    pl.kernel,
    out_type=jax.ShapeDtypeStruct((2, NACC, D), jnp.float32),
    mesh=_MESH,
    scratch_types=[
        pltpu.VMEM_SHARED((NACC, D), jnp.float32),
        pltpu.VMEM((4, CH, D), jnp.float32),
        pltpu.VMEM((8, 2, CH), jnp.int32),
        pltpu.VMEM((8, CH), jnp.float32),
        pltpu.SemaphoreType.DMA((8,)),
        pltpu.SemaphoreType.DMA((4,)),
        pltpu.SemaphoreType.DMA((4,)),
    ],
)
def _sc_edge(y_hbm, edata_hbm, w_hbm, out_hbm, acc, rows, ebuf, wbuf,
             csem, gsem, ssem):
    c = lax.axis_index("c")
    s = lax.axis_index("s")
    nch = jnp.where(c == 0, NCH_A, NCH_B)
    cbase = jnp.where(c == 0, s * NCH_A, 16 * NCH_A + s * NCH_B)

    def fire_c(k):
        q = k % 8
        pltpu.async_copy(edata_hbm.at[cbase + k], ebuf.at[q], csem.at[q])
        pltpu.async_copy(w_hbm.at[pl.ds((cbase + k) * CH, CH)], wbuf.at[q],
                         csem.at[q])

    def fire_g(k):
        pltpu.async_copy(y_hbm.at[ebuf.at[k % 8, 0]], rows.at[k % 4],
                         gsem.at[k % 4])

    def wait_c(q):
        pltpu.make_async_copy(edata_hbm.at[cbase], ebuf.at[q],
                              csem.at[q]).wait()
        pltpu.make_async_copy(w_hbm.at[pl.ds(0, CH)], wbuf.at[q],
                              csem.at[q]).wait()

    def wait_g(p):
        pltpu.make_async_copy(y_hbm.at[pl.ds(0, CH)], rows.at[p],
                              gsem.at[p]).wait()

    def wait_s(p):
        pltpu.make_async_copy(y_hbm.at[pl.ds(0, CH)], rows.at[p],
                              ssem.at[p]).wait()

    for j in range(5):
        fire_c(j)
    _zero_vmem_2d(rows.at[0], CH)

    def zchunk(j, _):
        pltpu.sync_copy(rows.at[0], acc.at[pl.ds(s * 640 + j * CH, CH)])
        return 0

    lax.fori_loop(0, 640 // CH, zchunk, 0)
    plsc.subcore_barrier()

    def body(k, _):
        p = k % 4
        q = k % 8
        p2 = (k + 2) % 4

        @pl.when(k >= 2)
        def _ws():
            wait_s(p2)

        @pl.when(k < nch - 2)
        def _fg():
            wait_c((k + 2) % 8)
            fire_g(k + 2)

        @pl.when(k < nch - 5)
        def _fc():
            fire_c(k + 5)

        wait_g(p)

        def scale(g, _):
            wv = wbuf[q, pl.ds(g * 16, 16)]
            for j in range(16):
                e = g * 16 + j
                w = jnp.broadcast_to(wv[j], (16,))
                for dd in range(D // 16):
                    sl = pl.ds(dd * 16, 16)
                    rows[p, e, sl] = rows[p, e, sl] * w
            return 0

        lax.fori_loop(0, CH // 16, scale, 0)
        pltpu.async_copy(rows.at[p], acc.at[ebuf.at[q, 1]], ssem.at[p],
                         add=True)
        return 0

    wait_c(0)
    fire_g(0)
    wait_c(1)
    fire_g(1)
    lax.fori_loop(0, nch, body, 0)
    wait_s((nch - 2) % 4)
    wait_s((nch - 1) % 4)
    plsc.subcore_barrier()

    pltpu.sync_copy(acc.at[pl.ds(s * 640, 640)],
                    out_hbm.at[c].at[pl.ds(s * 640, 640)])


def _y_body(comp_ref, x_ref, basis_ref, y_ref):
    r = pl.program_id(1)
    rows = lax.broadcasted_iota(jnp.int32, (R, 1), 0)
    comp_r = jnp.sum(jnp.where(rows == r, comp_ref[...], 0.0), axis=0,
                     keepdims=True)
    w_r = jnp.dot(comp_r, basis_ref[...].reshape(NB, D * D),
                  preferred_element_type=jnp.float32).reshape(D, D)
    y_ref[0] = jnp.dot(x_ref[...], w_r, preferred_element_type=jnp.float32)


def _y_kernel(x, basis, comp):
    return pl.pallas_call(
        _y_body,
        grid=(N // BN, R),
        in_specs=[
            pl.BlockSpec((R, NB), lambda n, r: (0, 0)),
            pl.BlockSpec((BN, D), lambda n, r: (n, 0)),
            pl.BlockSpec((NB, D, D), lambda n, r: (0, 0, 0)),
        ],
        out_specs=pl.BlockSpec((1, BN, D), lambda n, r: (r, n, 0)),
        out_shape=jax.ShapeDtypeStruct((R, N, D), jnp.float32),
    )(comp, x, basis)


def _combine_body(x_ref, agg_ref, root_ref, bias_ref, lnw_ref, lnb_ref, out_ref):
    agg = agg_ref[0] + agg_ref[1]
    t = agg + jnp.dot(x_ref[...], root_ref[...],
                      preferred_element_type=jnp.float32) + bias_ref[...]
    mu = jnp.mean(t, axis=-1, keepdims=True)
    var = jnp.mean((t - mu) ** 2, axis=-1, keepdims=True)
    t = (t - mu) * lax.rsqrt(var + 1e-5) * lnw_ref[...] + lnb_ref[...]
    out_ref[...] = x_ref[...] + jnp.maximum(t, 0.0)


def _combine_kernel(x, agg2, root, bias, lnw, lnb):
    return pl.pallas_call(
        _combine_body,
        grid=(N // BN,),
        in_specs=[
            pl.BlockSpec((BN, D), lambda n: (n, 0)),
            pl.BlockSpec((2, BN, D), lambda n: (0, n, 0)),
            pl.BlockSpec((D, D), lambda n: (0, 0)),
            pl.BlockSpec((1, D), lambda n: (0, 0)),
            pl.BlockSpec((1, D), lambda n: (0, 0)),
            pl.BlockSpec((1, D), lambda n: (0, 0)),
        ],
        out_specs=pl.BlockSpec((BN, D), lambda n: (n, 0)),
        out_shape=jax.ShapeDtypeStruct((N, D), jnp.float32),
    )(x, agg2, root, bias.reshape(1, D), lnw.reshape(1, D), lnb.reshape(1, D))


def _mean_body(x_ref, out_ref):
    @pl.when(pl.program_id(0) == 0)
    def _init():
        out_ref[...] = jnp.zeros_like(out_ref)
    out_ref[...] += jnp.sum(x_ref[...], axis=0, keepdims=True) * (1.0 / N)


def _mean_kernel(x):
    return pl.pallas_call(
        _mean_body,
        grid=(N // BN,),
        in_specs=[pl.BlockSpec((BN, D), lambda n: (n, 0))],
        out_specs=pl.BlockSpec((1, D), lambda n: (0, 0)),
        out_shape=jax.ShapeDtypeStruct((1, D), jnp.float32),
    )(x)


def kernel(node_ids, edge_index, edge_type, emb,
           basis0, comp0, root0, bias0, lnw0, lnb0,
           basis1, comp1, root1, bias1, lnw1, lnb1,
           basis2, comp2, root2, bias2, lnw2, lnb2):
    src = edge_index[0]
    dst = edge_index[1]
    etype = edge_type

    pad = EPAD - E
    seg = jnp.concatenate([dst * R + etype,
                           jnp.full((pad,), N * R, jnp.int32)])
    gidx = jnp.concatenate([etype * N + src, jnp.zeros((pad,), jnp.int32)])
    dstp = jnp.concatenate([dst, jnp.full((pad,), N, jnp.int32)])
    ids = jnp.concatenate([node_ids,
                           jnp.zeros((NPAD - N,), node_ids.dtype)])

    w_edge = _sc_weights(seg)
    x = _sc_x0(emb, ids)[:N]

    edata = jnp.stack([gidx.reshape(-1, CH), dstp.reshape(-1, CH)], axis=1)

    layers = [(basis0, comp0, root0, bias0, lnw0, lnb0),
              (basis1, comp1, root1, bias1, lnw1, lnb1),
              (basis2, comp2, root2, bias2, lnw2, lnb2)]
    for (ba, co, ro, bi, lw, lb) in layers:
        y = _y_kernel(x, ba, co).reshape(R * N, D)
        agg2 = _sc_edge(y, edata, w_edge)
        x = _combine_kernel(x, agg2, ro, bi, lw, lb)

    return (x, _mean_kernel(x))


# speedup vs baseline: 1.1865x; 1.1865x over previous
import functools

import jax
import jax.numpy as jnp
from jax import lax
from jax.experimental import pallas as pl
from jax.experimental.pallas import tpu as pltpu
from jax.experimental.pallas import tpu_sc as plsc

N = 10000
E = 320000
R = 16
NB = 16
D = 128
BN = 1000

NTILES = 32
CH = 80
EPT = 10240
EPAD = NTILES * EPT
EPC = EPAD // 16
NRP = 163840
NACC = 10240
NPAD = NTILES * 320

_MESH = plsc.VectorSubcoreMesh(core_axis_name="c", subcore_axis_name="s")


def _zero_vmem_1d(buf, nvec):
    z = jnp.zeros((16,), jnp.float32)

    def body(i, _):
        buf[pl.ds(i * 16, 16)] = z
        return 0

    lax.fori_loop(0, nvec, body, 0)


def _zero_vmem_2d(buf, nrows):
    z = jnp.zeros((16,), jnp.float32)

    def body(i, _):
        for dd in range(D // 16):
            buf[i, pl.ds(dd * 16, 16)] = z
        return 0

    lax.fori_loop(0, nrows, body, 0)


@functools.partial(
    pl.kernel,
    out_type=jax.ShapeDtypeStruct((NPAD, D), jnp.float32),
    mesh=_MESH,
    scratch_types=[
        pltpu.VMEM((80,), jnp.int32),
        pltpu.VMEM((80, D), jnp.float32),
        pltpu.SemaphoreType.DMA,
    ],
)
def _sc_x0(emb_hbm, ids_hbm, out_hbm, idxb, rows, sem):
    c = lax.axis_index("c")
    s = lax.axis_index("s")
    wid = c * 16 + s

    def chunk(j, _):
        base = wid * 320 + j * 80
        pltpu.sync_copy(ids_hbm.at[pl.ds(base, 80)], idxb)
        pltpu.async_copy(emb_hbm.at[idxb], rows, sem).wait()
        pltpu.sync_copy(rows, out_hbm.at[pl.ds(base, 80)])
        return 0

    lax.fori_loop(0, 4, chunk, 0)


@functools.partial(
    pl.kernel,
    out_type=jax.ShapeDtypeStruct((EPAD,), jnp.float32),
    mesh=_MESH,
    scratch_types=[
        pltpu.VMEM_SHARED((NRP,), jnp.float32),
        pltpu.VMEM((CH,), jnp.int32),
        pltpu.VMEM((CH,), jnp.float32),
        pltpu.VMEM((2048,), jnp.float32),
        pltpu.SemaphoreType.DMA,
    ],
)
def _sc_weights(seg_hbm, w_hbm, cnt, segb, wb, stage, sem):
    c = lax.axis_index("c")
    s = lax.axis_index("s")
    wid = c * 16 + s

    def ones_body(i, _):
        wb[pl.ds(i * 16, 16)] = jnp.full((16,), 1.0, jnp.float32)
        return 0

    lax.fori_loop(0, CH // 16, ones_body, 0)

    _zero_vmem_1d(stage, 128)

    def zchunk(j, _):
        pltpu.sync_copy(stage, cnt.at[pl.ds(s * (NRP // 16) + j * 2048, 2048)])
        return 0

    lax.fori_loop(0, NRP // 16 // 2048, zchunk, 0)
    plsc.subcore_barrier()

    def cchunk(k, _):
        pltpu.sync_copy(seg_hbm.at[pl.ds(s * EPC + k * CH, CH)], segb)
        pltpu.sync_copy(wb, cnt.at[segb], add=True)
        return 0

    lax.fori_loop(0, EPC // CH, cchunk, 0)
    plsc.subcore_barrier()

    def ichunk(j, _):
        base = s * (NRP // 16) + j * 2048
        pltpu.sync_copy(cnt.at[pl.ds(base, 2048)], stage)

        def inv(i, _):
            v = stage[pl.ds(i * 16, 16)]
            stage[pl.ds(i * 16, 16)] = 1.0 / jnp.maximum(v, 1.0)
            return 0

        lax.fori_loop(0, 128, inv, 0)
        pltpu.sync_copy(stage, cnt.at[pl.ds(base, 2048)])
        return 0

    lax.fori_loop(0, NRP // 16 // 2048, ichunk, 0)
    plsc.subcore_barrier()

    def gchunk(k, _):
        base = wid * EPT + k * CH
        pltpu.sync_copy(seg_hbm.at[pl.ds(base, CH)], segb)
        pltpu.async_copy(cnt.at[segb], wb, sem).wait()
        pltpu.sync_copy(wb, w_hbm.at[pl.ds(base, CH)])
        return 0

    lax.fori_loop(0, EPT // CH, gchunk, 0)


NCHT = EPT // CH
NCH_A = 151
NCH_B = 2 * NCHT - 151
RBYTES = CH * D * 4


@functools.partial(
    pl.kernel,
    out_type=jax.ShapeDtypeStruct((2, NACC, D), jnp.float32),
    mesh=_MESH,
    scratch_types=[
        pltpu.VMEM_SHARED((NACC, D), jnp.float32),
        pltpu.VMEM((4, CH, D), jnp.float32),
        pltpu.VMEM((8, 2, CH), jnp.int32),
        pltpu.VMEM((8, CH), jnp.float32),
        pltpu.SemaphoreType.DMA((8,)),
        pltpu.SemaphoreType.DMA((4,)),
        pltpu.SemaphoreType.DMA((4,)),
    ],
)
def _sc_edge(y_hbm, edata_hbm, w_hbm, out_hbm, acc, rows, ebuf, wbuf,
             csem, gsem, ssem):
    c = lax.axis_index("c")
    s = lax.axis_index("s")
    nch = jnp.where(c == 0, NCH_A, NCH_B)
    cbase = jnp.where(c == 0, s * NCH_A, 16 * NCH_A + s * NCH_B)

    def fire_c(k):
        q = k % 8
        pltpu.async_copy(edata_hbm.at[cbase + k], ebuf.at[q], csem.at[q])
        pltpu.async_copy(w_hbm.at[pl.ds((cbase + k) * CH, CH)], wbuf.at[q],
                         csem.at[q])

    def fire_g(k):
        pltpu.async_copy(y_hbm.at[ebuf.at[k % 8, 0]], rows.at[k % 4],
                         gsem.at[k % 4])

    def wait_c(q):
        pltpu.make_async_copy(edata_hbm.at[cbase], ebuf.at[q],
                              csem.at[q]).wait()
        pltpu.make_async_copy(w_hbm.at[pl.ds(0, CH)], wbuf.at[q],
                              csem.at[q]).wait()

    def wait_g(p):
        pltpu.make_async_copy(y_hbm.at[pl.ds(0, CH)], rows.at[p],
                              gsem.at[p]).wait()

    def wait_s(p):
        pltpu.make_async_copy(y_hbm.at[pl.ds(0, CH)], rows.at[p],
                              ssem.at[p]).wait()

    for j in range(5):
        fire_c(j)
    _zero_vmem_2d(rows.at[0], CH)

    def zchunk(j, _):
        pltpu.sync_copy(rows.at[0], acc.at[pl.ds(s * 640 + j * CH, CH)])
        return 0

    lax.fori_loop(0, 640 // CH, zchunk, 0)
    plsc.subcore_barrier()

    def body(k, _):
        p = k % 4
        q = k % 8
        p2 = (k + 2) % 4

        @pl.when(k >= 2)
        def _ws():
            wait_s(p2)

        @pl.when(k < nch - 2)
        def _fg():
            wait_c((k + 2) % 8)
            fire_g(k + 2)

        @pl.when(k < nch - 5)
        def _fc():
            fire_c(k + 5)

        wait_g(p)

        def scale(g, _):
            wv = wbuf[q, pl.ds(g * 16, 16)]
            for j in range(16):
                e = g * 16 + j
                w = jnp.broadcast_to(wv[j], (16,))
                for dd in range(D // 16):
                    sl = pl.ds(dd * 16, 16)
                    rows[p, e, sl] = rows[p, e, sl] * w
            return 0

        lax.fori_loop(0, CH // 16, scale, 0)
        pltpu.async_copy(rows.at[p], acc.at[ebuf.at[q, 1]], ssem.at[p],
                         add=True)
        return 0

    wait_c(0)
    fire_g(0)
    wait_c(1)
    fire_g(1)
    lax.fori_loop(0, nch, body, 0)
    wait_s((nch - 2) % 4)
    wait_s((nch - 1) % 4)
    plsc.subcore_barrier()

    pltpu.sync_copy(acc.at[pl.ds(s * 640, 640)],
                    out_hbm.at[c].at[pl.ds(s * 640, 640)])


def _y_body(comp_ref, x_ref, basis_ref, y_ref):
    r = pl.program_id(1)
    rows = lax.broadcasted_iota(jnp.int32, (R, 1), 0)
    comp_r = jnp.sum(jnp.where(rows == r, comp_ref[...], 0.0), axis=0,
                     keepdims=True)
    w_r = jnp.dot(comp_r, basis_ref[...].reshape(NB, D * D),
                  preferred_element_type=jnp.float32).reshape(D, D)
    y_ref[0] = jnp.dot(x_ref[...], w_r, preferred_element_type=jnp.float32)


def _y_kernel(x, basis, comp):
    return pl.pallas_call(
        _y_body,
        grid=(N // BN, R),
        in_specs=[
            pl.BlockSpec((R, NB), lambda n, r: (0, 0)),
            pl.BlockSpec((BN, D), lambda n, r: (n, 0)),
            pl.BlockSpec((NB, D, D), lambda n, r: (0, 0, 0)),
        ],
        out_specs=pl.BlockSpec((1, BN, D), lambda n, r: (r, n, 0)),
        out_shape=jax.ShapeDtypeStruct((R, N, D), jnp.float32),
    )(comp, x, basis)


def _combine_body(x_ref, agg_ref, root_ref, bias_ref, lnw_ref, lnb_ref, out_ref):
    agg = agg_ref[0] + agg_ref[1]
    t = agg + jnp.dot(x_ref[...], root_ref[...],
                      preferred_element_type=jnp.float32) + bias_ref[...]
    mu = jnp.mean(t, axis=-1, keepdims=True)
    var = jnp.mean((t - mu) ** 2, axis=-1, keepdims=True)
    t = (t - mu) * lax.rsqrt(var + 1e-5) * lnw_ref[...] + lnb_ref[...]
    out_ref[...] = x_ref[...] + jnp.maximum(t, 0.0)


def _combine_kernel(x, agg2, root, bias, lnw, lnb):
    return pl.pallas_call(
        _combine_body,
        grid=(N // BN,),
        in_specs=[
            pl.BlockSpec((BN, D), lambda n: (n, 0)),
            pl.BlockSpec((2, BN, D), lambda n: (0, n, 0)),
            pl.BlockSpec((D, D), lambda n: (0, 0)),
            pl.BlockSpec((1, D), lambda n: (0, 0)),
            pl.BlockSpec((1, D), lambda n: (0, 0)),
            pl.BlockSpec((1, D), lambda n: (0, 0)),
        ],
        out_specs=pl.BlockSpec((BN, D), lambda n: (n, 0)),
        out_shape=jax.ShapeDtypeStruct((N, D), jnp.float32),
    )(x, agg2, root, bias.reshape(1, D), lnw.reshape(1, D), lnb.reshape(1, D))


def _mean_body(x_ref, out_ref):
    @pl.when(pl.program_id(0) == 0)
    def _init():
        out_ref[...] = jnp.zeros_like(out_ref)
    out_ref[...] += jnp.sum(x_ref[...], axis=0, keepdims=True) * (1.0 / N)


def _mean_kernel(x):
    return pl.pallas_call(
        _mean_body,
        grid=(N // BN,),
        in_specs=[pl.BlockSpec((BN, D), lambda n: (n, 0))],
        out_specs=pl.BlockSpec((1, D), lambda n: (0, 0)),
        out_shape=jax.ShapeDtypeStruct((1, D), jnp.float32),
    )(x)


def kernel(node_ids, edge_index, edge_type, emb,
           basis0, comp0, root0, bias0, lnw0, lnb0,
           basis1, comp1, root1, bias1, lnw1, lnb1,
           basis2, comp2, root2, bias2, lnw2, lnb2):
    src = edge_index[0]
    dst = edge_index[1]
    etype = edge_type

    pad = EPAD - E
    seg = jnp.concatenate([dst * R + etype,
                           jnp.full((pad,), N * R, jnp.int32)])
    gidx = jnp.concatenate([etype * N + src, jnp.zeros((pad,), jnp.int32)])
    dstp = jnp.concatenate([dst, jnp.full((pad,), N, jnp.int32)])
    ids = jnp.concatenate([node_ids,
                           jnp.zeros((NPAD - N,), node_ids.dtype)])

    w_edge = _sc_weights(seg)
    x = _sc_x0(emb, ids)[:N]

    edata = jnp.stack([gidx.reshape(-1, CH), dstp.reshape(-1, CH)], axis=1)

    layers = [(basis0, comp0, root0, bias0, lnw0, lnb0),
              (basis1, comp1, root1, bias1, lnw1, lnb1),
              (basis2, comp2, root2, bias2, lnw2, lnb2)]
    for (ba, co, ro, bi, lw, lb) in layers:
        y = _y_kernel(x, ba, co).reshape(R * N, D)
        agg2 = _sc_edge(y, edata, w_edge)
        x = _combine_kernel(x, agg2, ro, bi, lw, lb)

    return (x, _mean_kernel(x))


# speedup vs baseline: 1.2701x; 1.0704x over previous
import functools

import jax
import jax.numpy as jnp
from jax import lax
from jax.experimental import pallas as pl
from jax.experimental.pallas import tpu as pltpu
from jax.experimental.pallas import tpu_sc as plsc

N = 10000
E = 320000
R = 16
NB = 16
D = 128
BN = 1000

NTILES = 32
CH = 80
EPT = 10240
EPAD = NTILES * EPT
EPC = EPAD // 16
NRP = 163840
NACC = 10240
NPAD = NTILES * 320

_MESH = plsc.VectorSubcoreMesh(core_axis_name="c", subcore_axis_name="s")


def _zero_vmem_1d(buf, nvec):
    z = jnp.zeros((16,), jnp.float32)

    def body(i, _):
        buf[pl.ds(i * 16, 16)] = z
        return 0

    lax.fori_loop(0, nvec, body, 0)


def _zero_vmem_2d(buf, nrows):
    z = jnp.zeros((16,), jnp.float32)

    def body(i, _):
        for dd in range(D // 16):
            buf[i, pl.ds(dd * 16, 16)] = z
        return 0

    lax.fori_loop(0, nrows, body, 0)


@functools.partial(
    pl.kernel,
    out_type=jax.ShapeDtypeStruct((NPAD, D), jnp.float32),
    mesh=_MESH,
    scratch_types=[
        pltpu.VMEM((80,), jnp.int32),
        pltpu.VMEM((80, D), jnp.float32),
        pltpu.SemaphoreType.DMA,
    ],
)
def _sc_x0(emb_hbm, ids_hbm, out_hbm, idxb, rows, sem):
    c = lax.axis_index("c")
    s = lax.axis_index("s")
    wid = c * 16 + s

    def chunk(j, _):
        base = wid * 320 + j * 80
        pltpu.sync_copy(ids_hbm.at[pl.ds(base, 80)], idxb)
        pltpu.async_copy(emb_hbm.at[idxb], rows, sem).wait()
        pltpu.sync_copy(rows, out_hbm.at[pl.ds(base, 80)])
        return 0

    lax.fori_loop(0, 4, chunk, 0)


@functools.partial(
    pl.kernel,
    out_type=jax.ShapeDtypeStruct((EPAD,), jnp.float32),
    mesh=_MESH,
    scratch_types=[
        pltpu.VMEM_SHARED((NRP,), jnp.float32),
        pltpu.VMEM((CH,), jnp.int32),
        pltpu.VMEM((CH,), jnp.float32),
        pltpu.VMEM((2048,), jnp.float32),
        pltpu.SemaphoreType.DMA,
    ],
)
def _sc_weights(seg_hbm, w_hbm, cnt, segb, wb, stage, sem):
    c = lax.axis_index("c")
    s = lax.axis_index("s")
    wid = c * 16 + s

    def ones_body(i, _):
        wb[pl.ds(i * 16, 16)] = jnp.full((16,), 1.0, jnp.float32)
        return 0

    lax.fori_loop(0, CH // 16, ones_body, 0)

    _zero_vmem_1d(stage, 128)

    def zchunk(j, _):
        pltpu.sync_copy(stage, cnt.at[pl.ds(s * (NRP // 16) + j * 2048, 2048)])
        return 0

    lax.fori_loop(0, NRP // 16 // 2048, zchunk, 0)
    plsc.subcore_barrier()

    def cchunk(k, _):
        pltpu.sync_copy(seg_hbm.at[pl.ds(s * EPC + k * CH, CH)], segb)
        pltpu.sync_copy(wb, cnt.at[segb], add=True)
        return 0

    lax.fori_loop(0, EPC // CH, cchunk, 0)
    plsc.subcore_barrier()

    def ichunk(j, _):
        base = s * (NRP // 16) + j * 2048
        pltpu.sync_copy(cnt.at[pl.ds(base, 2048)], stage)

        def inv(i, _):
            v = stage[pl.ds(i * 16, 16)]
            stage[pl.ds(i * 16, 16)] = 1.0 / jnp.maximum(v, 1.0)
            return 0

        lax.fori_loop(0, 128, inv, 0)
        pltpu.sync_copy(stage, cnt.at[pl.ds(base, 2048)])
        return 0

    lax.fori_loop(0, NRP // 16 // 2048, ichunk, 0)
    plsc.subcore_barrier()

    def gchunk(k, _):
        base = wid * EPT + k * CH
        pltpu.sync_copy(seg_hbm.at[pl.ds(base, CH)], segb)
        pltpu.async_copy(cnt.at[segb], wb, sem).wait()
        pltpu.sync_copy(wb, w_hbm.at[pl.ds(base, CH)])
        return 0

    lax.fori_loop(0, EPT // CH, gchunk, 0)


NCHT = EPT // CH
NCH_A = 151
NCH_B = 2 * NCHT - 151
RBYTES = CH * D * 4


@functools.partial(
    pl.kernel,
    out_type=jax.ShapeDtypeStruct((2, NACC, D), jnp.float32),
    mesh=_MESH,
    scratch_types=[
        pltpu.VMEM_SHARED((NACC, D), jnp.float32),
        pltpu.VMEM((4, CH, D), jnp.float32),
        pltpu.VMEM((8, 2, CH), jnp.int32),
        pltpu.VMEM((8, CH), jnp.float32),
        pltpu.SemaphoreType.DMA((8,)),
        pltpu.SemaphoreType.DMA((4,)),
        pltpu.SemaphoreType.DMA((4,)),
    ],
)
def _sc_edge(y_hbm, edata_hbm, w_hbm, out_hbm, acc, rows, ebuf, wbuf,
             csem, gsem, ssem):
    c = lax.axis_index("c")
    s = lax.axis_index("s")
    nch = jnp.where(c == 0, NCH_A, NCH_B)
    cbase = jnp.where(c == 0, s * NCH_A, 16 * NCH_A + s * NCH_B)

    def fire_c(k):
        q = k % 8
        pltpu.async_copy(edata_hbm.at[cbase + k], ebuf.at[q], csem.at[q])
        pltpu.async_copy(w_hbm.at[pl.ds((cbase + k) * CH, CH)], wbuf.at[q],
                         csem.at[q])

    def fire_g(k):
        pltpu.async_copy(y_hbm.at[ebuf.at[k % 8, 0]], rows.at[k % 4],
                         gsem.at[k % 4])

    def wait_c(q):
        pltpu.make_async_copy(edata_hbm.at[cbase], ebuf.at[q],
                              csem.at[q]).wait()
        pltpu.make_async_copy(w_hbm.at[pl.ds(0, CH)], wbuf.at[q],
                              csem.at[q]).wait()

    def wait_g(p):
        pltpu.make_async_copy(y_hbm.at[pl.ds(0, CH)], rows.at[p],
                              gsem.at[p]).wait()

    def wait_s(p):
        pltpu.make_async_copy(y_hbm.at[pl.ds(0, CH)], rows.at[p],
                              ssem.at[p]).wait()

    for j in range(5):
        fire_c(j)
    _zero_vmem_2d(rows.at[0], CH)

    def zchunk(j, _):
        pltpu.sync_copy(rows.at[0], acc.at[pl.ds(s * 640 + j * CH, CH)])
        return 0

    lax.fori_loop(0, 640 // CH, zchunk, 0)
    plsc.subcore_barrier()

    def body(k, _):
        p = k % 4
        q = k % 8
        p2 = (k + 2) % 4

        @pl.when(k >= 2)
        def _ws():
            wait_s(p2)

        @pl.when(k < nch - 2)
        def _fg():
            wait_c((k + 2) % 8)
            fire_g(k + 2)

        @pl.when(k < nch - 5)
        def _fc():
            fire_c(k + 5)

        wait_g(p)

        def scale(g, _):
            wv = wbuf[q, pl.ds(g * 16, 16)]
            for j in range(16):
                e = g * 16 + j
                w = jnp.broadcast_to(wv[j], (16,))
                for dd in range(D // 16):
                    sl = pl.ds(dd * 16, 16)
                    rows[p, e, sl] = rows[p, e, sl] * w
            return 0

        lax.fori_loop(0, CH // 16, scale, 0)
        pltpu.async_copy(rows.at[p], acc.at[ebuf.at[q, 1]], ssem.at[p],
                         add=True)
        return 0

    wait_c(0)
    fire_g(0)
    wait_c(1)
    fire_g(1)
    lax.fori_loop(0, nch, body, 0)
    wait_s((nch - 2) % 4)
    wait_s((nch - 1) % 4)
    plsc.subcore_barrier()

    pltpu.sync_copy(acc.at[pl.ds(s * 640, 640)],
                    out_hbm.at[c].at[pl.ds(s * 640, 640)])


def _w_body(comp_ref, basis_ref, w_ref):
    w_ref[...] = jnp.dot(comp_ref[...], basis_ref[...].reshape(NB, D * D),
                         preferred_element_type=jnp.float32)


def _w_kernel(basis, comp):
    return pl.pallas_call(
        _w_body,
        out_shape=jax.ShapeDtypeStruct((R, D * D), jnp.float32),
    )(comp, basis)


def _y_body(x_ref, w_ref, y_ref):
    x = x_ref[...]
    for r in range(R):
        w_r = w_ref[r].reshape(D, D)
        y_ref[r] = jnp.dot(x, w_r, preferred_element_type=jnp.float32)


def _y_kernel(x, w_all):
    return pl.pallas_call(
        _y_body,
        grid=(N // BN,),
        in_specs=[
            pl.BlockSpec((BN, D), lambda n: (n, 0)),
            pl.BlockSpec((R, D * D), lambda n: (0, 0)),
        ],
        out_specs=pl.BlockSpec((R, BN, D), lambda n: (0, n, 0)),
        out_shape=jax.ShapeDtypeStruct((R, N, D), jnp.float32),
    )(x, w_all)


def _combine_body(x_ref, agg_ref, root_ref, bias_ref, lnw_ref, lnb_ref, out_ref):
    agg = agg_ref[0] + agg_ref[1]
    t = agg + jnp.dot(x_ref[...], root_ref[...],
                      preferred_element_type=jnp.float32) + bias_ref[...]
    mu = jnp.mean(t, axis=-1, keepdims=True)
    var = jnp.mean((t - mu) ** 2, axis=-1, keepdims=True)
    t = (t - mu) * lax.rsqrt(var + 1e-5) * lnw_ref[...] + lnb_ref[...]
    out_ref[...] = x_ref[...] + jnp.maximum(t, 0.0)


def _combine_kernel(x, agg2, root, bias, lnw, lnb):
    return pl.pallas_call(
        _combine_body,
        grid=(N // BN,),
        in_specs=[
            pl.BlockSpec((BN, D), lambda n: (n, 0)),
            pl.BlockSpec((2, BN, D), lambda n: (0, n, 0)),
            pl.BlockSpec((D, D), lambda n: (0, 0)),
            pl.BlockSpec((1, D), lambda n: (0, 0)),
            pl.BlockSpec((1, D), lambda n: (0, 0)),
            pl.BlockSpec((1, D), lambda n: (0, 0)),
        ],
        out_specs=pl.BlockSpec((BN, D), lambda n: (n, 0)),
        out_shape=jax.ShapeDtypeStruct((N, D), jnp.float32),
    )(x, agg2, root, bias.reshape(1, D), lnw.reshape(1, D), lnb.reshape(1, D))


def _mean_body(x_ref, out_ref):
    @pl.when(pl.program_id(0) == 0)
    def _init():
        out_ref[...] = jnp.zeros_like(out_ref)
    out_ref[...] += jnp.sum(x_ref[...], axis=0, keepdims=True) * (1.0 / N)


def _mean_kernel(x):
    return pl.pallas_call(
        _mean_body,
        grid=(N // BN,),
        in_specs=[pl.BlockSpec((BN, D), lambda n: (n, 0))],
        out_specs=pl.BlockSpec((1, D), lambda n: (0, 0)),
        out_shape=jax.ShapeDtypeStruct((1, D), jnp.float32),
    )(x)


def kernel(node_ids, edge_index, edge_type, emb,
           basis0, comp0, root0, bias0, lnw0, lnb0,
           basis1, comp1, root1, bias1, lnw1, lnb1,
           basis2, comp2, root2, bias2, lnw2, lnb2):
    src = edge_index[0]
    dst = edge_index[1]
    etype = edge_type

    pad = EPAD - E
    seg = jnp.concatenate([dst * R + etype,
                           jnp.full((pad,), N * R, jnp.int32)])
    gidx = jnp.concatenate([etype * N + src, jnp.zeros((pad,), jnp.int32)])
    dstp = jnp.concatenate([dst, jnp.full((pad,), N, jnp.int32)])
    ids = jnp.concatenate([node_ids,
                           jnp.zeros((NPAD - N,), node_ids.dtype)])

    w_edge = _sc_weights(seg)
    x = _sc_x0(emb, ids)[:N]

    edata = jnp.stack([gidx.reshape(-1, CH), dstp.reshape(-1, CH)], axis=1)

    layers = [(basis0, comp0, root0, bias0, lnw0, lnb0),
              (basis1, comp1, root1, bias1, lnw1, lnb1),
              (basis2, comp2, root2, bias2, lnw2, lnb2)]
    for (ba, co, ro, bi, lw, lb) in layers:
        y = _y_kernel(x, _w_kernel(ba, co)).reshape(R * N, D)
        agg2 = _sc_edge(y, edata, w_edge)
        x = _combine_kernel(x, agg2, ro, bi, lw, lb)

    return (x, _mean_kernel(x))
